# baseline (speedup 1.0000x reference)
import functools

import jax
import jax.numpy as jnp
from jax import lax
from jax.experimental import pallas as pl
from jax.experimental.pallas import tpu as pltpu
from jax.experimental.pallas import tpu_sc as plsc

N = 10000
EH = 5000
NNZ = 320000
HID = 128
OUT_C = 16

NC = 2
NS = 16
NW = NC * NS
EPW = NNZ // NW


def _mesh():
    return plsc.VectorSubcoreMesh(core_axis_name="c", subcore_axis_name="s")


_CH = 200


def _zero_acc(z_h, acc, sid, U):
    def zc(k, _):
        @pl.when(k % NS == sid)
        def _():
            pltpu.sync_copy(z_h.at[pl.ds(k * _CH, _CH)],
                            acc.at[pl.ds(k * _CH, _CH)])
        return 0
    lax.fori_loop(0, U // _CH, zc, 0)


def _acc_out(acc, out_h, cid, sid, U):
    def oc(k, _):
        @pl.when(k % NS == sid)
        def _():
            pltpu.sync_copy(acc.at[pl.ds(k * _CH, _CH)],
                            out_h.at[cid, pl.ds(k * _CH, _CH)])
        return 0
    lax.fori_loop(0, U // _CH, oc, 0)


def _sc_hist(src, dst):
    B = 400

    @functools.partial(
        pl.kernel,
        out_type=(jax.ShapeDtypeStruct((NC, EH), jnp.float32),
                  jax.ShapeDtypeStruct((NC, N), jnp.float32)),
        mesh=_mesh(),
        scratch_types=[pltpu.VMEM((B,), jnp.int32),
                       pltpu.VMEM((B,), jnp.int32),
                       pltpu.VMEM((B,), jnp.float32),
                       pltpu.VMEM_SHARED((EH,), jnp.float32),
                       pltpu.VMEM_SHARED((N,), jnp.float32)],
    )
    def k(src_h, dst_h, zeh_h, zn_h, de_h, dn_h, srcb, dstb, onesb, eacc, nacc):
        cid = lax.axis_index("c")
        sid = lax.axis_index("s")
        wid = sid * NC + cid

        @pl.when(sid == 0)
        def _():
            pltpu.sync_copy(zeh_h, eacc)
            pltpu.sync_copy(zn_h, nacc)

        def fill(i, _):
            onesb[pl.ds(i * 16, 16)] = jnp.full((16,), 1.0, jnp.float32)
            return 0
        lax.fori_loop(0, B // 16, fill, 0)
        plsc.subcore_barrier()

        def body(i, _):
            base = wid * EPW + i * B
            pltpu.sync_copy(dst_h.at[pl.ds(base, B)], dstb)
            pltpu.sync_copy(src_h.at[pl.ds(base, B)], srcb)
            pltpu.sync_copy(onesb, eacc.at[dstb], add=True)
            pltpu.sync_copy(onesb, nacc.at[srcb], add=True)
            return 0
        lax.fori_loop(0, EPW // B, body, 0)
        plsc.subcore_barrier()

        @pl.when(sid == 0)
        def _():
            pltpu.sync_copy(eacc, de_h.at[cid])
            pltpu.sync_copy(nacc, dn_h.at[cid])

    return k(src, dst, jnp.zeros((EH,), jnp.float32), jnp.zeros((N,), jnp.float32))


def _sc_gather(table, idx, W):
    B = 51200 // W
    nb = EPW // B

    @functools.partial(
        pl.kernel,
        out_type=jax.ShapeDtypeStruct((NNZ, W), jnp.float32),
        mesh=_mesh(),
        scratch_types=[pltpu.VMEM((B,), jnp.int32),
                       pltpu.VMEM((B, W), jnp.float32),
                       pltpu.SemaphoreType.DMA],
    )
    def k(table_h, idx_h, out_h, idxb, rows, sem):
        cid = lax.axis_index("c")
        sid = lax.axis_index("s")
        wid = sid * NC + cid

        def body(i, _):
            base = wid * EPW + i * B
            pltpu.sync_copy(idx_h.at[pl.ds(base, B)], idxb)
            pltpu.async_copy(table_h.at[idxb], rows, sem).wait()
            pltpu.sync_copy(rows, out_h.at[pl.ds(base, B)])
            return 0
        lax.fori_loop(0, nb, body, 0)

    return k(table, idx)


def _sc_gs(table, gidx, didx, U, W, scale=None):
    scaled = scale is not None
    B = 200 if (U == EH and not scaled) else 80
    nb = EPW // B
    outs = [jax.ShapeDtypeStruct((NC, U, W), jnp.float32)]
    scr = ([pltpu.VMEM((B,), jnp.int32)] * 4 +
           [pltpu.VMEM((B, W), jnp.float32)] * 2 +
           [pltpu.VMEM_SHARED((U, W), jnp.float32)] +
           [pltpu.SemaphoreType.DMA] * 2)
    ins = [table, gidx, didx, jnp.zeros((U, W), jnp.float32)]
    if scaled:
        outs.append(jax.ShapeDtypeStruct((NC, U), jnp.float32))
        scr += [pltpu.VMEM((B,), jnp.float32)] * 2 + \
               [pltpu.VMEM_SHARED((U,), jnp.float32)]
        ins += [jnp.zeros((U,), jnp.float32), scale]

    @functools.partial(pl.kernel,
                       out_type=tuple(outs) if scaled else outs[0],
                       mesh=_mesh(), scratch_types=scr)
    def k(*refs):
        if scaled:
            (table_h, gidx_h, didx_h, z_h, zu_h, sc_h, out_h, zout_h,
             gb0, gb1, db0, db1, rows0, rows1, acc, sem0, sem1,
             scb0, scb1, zacc) = refs
        else:
            (table_h, gidx_h, didx_h, z_h, out_h,
             gb0, gb1, db0, db1, rows0, rows1, acc, sem0, sem1) = refs
            sc_h = zu_h = zout_h = scb0 = scb1 = zacc = None
        cid = lax.axis_index("c")
        sid = lax.axis_index("s")
        wid = sid * NC + cid

        _zero_acc(z_h, acc, sid, U)
        if scaled:
            @pl.when(sid == 0)
            def _():
                pltpu.sync_copy(zu_h, zacc)
        plsc.subcore_barrier()

        def start(base, gb, db, scb, rows, sem):
            pltpu.sync_copy(gidx_h.at[pl.ds(base, B)], gb)
            pltpu.sync_copy(didx_h.at[pl.ds(base, B)], db)
            if scaled:
                pltpu.sync_copy(sc_h.at[pl.ds(base, B)], scb)
            return pltpu.async_copy(table_h.at[gb], rows, sem)

        def finish(d, db, scb, rows):
            d.wait()
            if scaled:
                def mule(t, _):
                    svec = scb[pl.ds(16 * t, 16)]
                    for j in range(16):
                        sv = svec[j]
                        row = rows.at[t * 16 + j]
                        for v in range(W // 16):
                            row[pl.ds(16 * v, 16)] = row[pl.ds(16 * v, 16)] * sv
                    return 0
                lax.fori_loop(0, B // 16, mule, 0)
            pltpu.sync_copy(rows, acc.at[db], add=True)
            if scaled:
                pltpu.sync_copy(scb, zacc.at[db], add=True)

        def body(j, _):
            b0 = wid * EPW + (2 * j) * B
            d0 = start(b0, gb0, db0, scb0, rows0, sem0)
            d1 = start(b0 + B, gb1, db1, scb1, rows1, sem1)
            finish(d0, db0, scb0, rows0)
            finish(d1, db1, scb1, rows1)
            return 0
        lax.fori_loop(0, nb // 2, body, 0)
        if nb % 2:
            bl = wid * EPW + (nb - 1) * B
            finish(start(bl, gb0, db0, scb0, rows0, sem0), db0, scb0, rows0)

        plsc.subcore_barrier()
        _acc_out(acc, out_h, cid, sid, U)
        if scaled:
            @pl.when(sid == 0)
            def _():
                pltpu.sync_copy(zacc, zout_h.at[cid])

    return k(*ins)


def _sc_gather3(tA, iA, tB, iB, tC, iC):
    B = 200
    nb = EPW // B
    sh = jax.ShapeDtypeStruct((NNZ, HID), jnp.float32)

    @functools.partial(
        pl.kernel, out_type=(sh, sh, sh), mesh=_mesh(),
        scratch_types=[pltpu.VMEM((B,), jnp.int32)] * 3 +
                      [pltpu.VMEM((B, HID), jnp.float32)] * 3 +
                      [pltpu.SemaphoreType.DMA] * 6,
    )
    def k(tA_h, iA_h, tB_h, iB_h, tC_h, iC_h, oA_h, oB_h, oC_h,
          ia, ib, ic, ra, rb, rc, s0, s1, s2, s3, s4, s5):
        cid = lax.axis_index("c")
        sid = lax.axis_index("s")
        wid = sid * NC + cid

        def body(j, _):
            b0 = wid * EPW + j * B
            pltpu.sync_copy(iA_h.at[pl.ds(b0, B)], ia)
            pltpu.sync_copy(iB_h.at[pl.ds(b0, B)], ib)
            pltpu.sync_copy(iC_h.at[pl.ds(b0, B)], ic)
            dA = pltpu.async_copy(tA_h.at[ia], ra, s0)
            dB = pltpu.async_copy(tB_h.at[ib], rb, s1)
            dC = pltpu.async_copy(tC_h.at[ic], rc, s2)
            dA.wait()
            wA = pltpu.async_copy(ra, oA_h.at[pl.ds(b0, B)], s3)
            dB.wait()
            wB = pltpu.async_copy(rb, oB_h.at[pl.ds(b0, B)], s4)
            dC.wait()
            wC = pltpu.async_copy(rc, oC_h.at[pl.ds(b0, B)], s5)
            wA.wait()
            wB.wait()
            wC.wait()
            return 0
        lax.fori_loop(0, nb, body, 0)

    return k(tA, iA, tB, iB, tC, iC)


def _sc_scatter(sg, scale, didx, U):
    B = 200 if U == EH else 80
    nb = EPW // B
    outs = (jax.ShapeDtypeStruct((NC, U, HID), jnp.float32),
            jax.ShapeDtypeStruct((NC, U), jnp.float32))
    scr = ([pltpu.VMEM((B,), jnp.int32)] * 2 +
           [pltpu.VMEM((B, HID), jnp.float32)] * 2 +
           [pltpu.VMEM((B,), jnp.float32)] * 2 +
           [pltpu.VMEM_SHARED((U, HID), jnp.float32),
            pltpu.VMEM_SHARED((U,), jnp.float32)] +
           [pltpu.SemaphoreType.DMA] * 2)

    @functools.partial(pl.kernel, out_type=outs, mesh=_mesh(),
                       scratch_types=scr)
    def k(sg_h, sc_h, didx_h, z_h, zu_h, out_h, zout_h,
          db0, db1, rows0, rows1, scb0, scb1, acc, zacc, sem0, sem1):
        cid = lax.axis_index("c")
        sid = lax.axis_index("s")
        wid = sid * NC + cid

        _zero_acc(z_h, acc, sid, U)

        @pl.when(sid == 0)
        def _():
            pltpu.sync_copy(zu_h, zacc)
        plsc.subcore_barrier()

        def start(base, db, scb, rows, sem):
            pltpu.sync_copy(didx_h.at[pl.ds(base, B)], db)
            pltpu.sync_copy(sc_h.at[pl.ds(base, B)], scb)
            return pltpu.async_copy(sg_h.at[pl.ds(base, B)], rows, sem)

        def finish(d, db, scb, rows):
            d.wait()
            pltpu.sync_copy(rows, acc.at[db], add=True)
            pltpu.sync_copy(scb, zacc.at[db], add=True)

        def body(j, _):
            b0 = wid * EPW + (2 * j) * B
            d0 = start(b0, db0, scb0, rows0, sem0)
            d1 = start(b0 + B, db1, scb1, rows1, sem1)
            finish(d0, db0, scb0, rows0)
            finish(d1, db1, scb1, rows1)
            return 0
        lax.fori_loop(0, nb // 2, body, 0)
        if nb % 2:
            bl = wid * EPW + (nb - 1) * B
            finish(start(bl, db0, scb0, rows0, sem0), db0, scb0, rows0)

        plsc.subcore_barrier()
        _acc_out(acc, out_h, cid, sid, U)

        @pl.when(sid == 0)
        def _():
            pltpu.sync_copy(zacc, zout_h.at[cid])

    return k(sg, scale, didx, jnp.zeros((U, HID), jnp.float32),
             jnp.zeros((U,), jnp.float32))


def _sc_gather2(tA, iA, tB, iB):
    B = 200
    nb2 = EPW // (2 * B)
    sh = jax.ShapeDtypeStruct((NNZ, HID), jnp.float32)

    @functools.partial(
        pl.kernel, out_type=(sh, sh), mesh=_mesh(),
        scratch_types=[pltpu.VMEM((B,), jnp.int32)] * 4 +
                      [pltpu.VMEM((B, HID), jnp.float32)] * 4 +
                      [pltpu.SemaphoreType.DMA] * 8,
    )
    def k(tA_h, iA_h, tB_h, iB_h, oA_h, oB_h,
          ia0, ia1, ib0, ib1, ra0, ra1, rb0, rb1,
          s0, s1, s2, s3, s4, s5, s6, s7):
        cid = lax.axis_index("c")
        sid = lax.axis_index("s")
        wid = sid * NC + cid

        def body(j, _):
            b0 = wid * EPW + (2 * j) * B
            b1 = b0 + B
            pltpu.sync_copy(iA_h.at[pl.ds(b0, B)], ia0)
            pltpu.sync_copy(iB_h.at[pl.ds(b0, B)], ib0)
            pltpu.sync_copy(iA_h.at[pl.ds(b1, B)], ia1)
            pltpu.sync_copy(iB_h.at[pl.ds(b1, B)], ib1)
            dA0 = pltpu.async_copy(tA_h.at[ia0], ra0, s0)
            dB0 = pltpu.async_copy(tB_h.at[ib0], rb0, s1)
            dA1 = pltpu.async_copy(tA_h.at[ia1], ra1, s2)
            dB1 = pltpu.async_copy(tB_h.at[ib1], rb1, s3)
            dA0.wait()
            wA0 = pltpu.async_copy(ra0, oA_h.at[pl.ds(b0, B)], s4)
            dB0.wait()
            wB0 = pltpu.async_copy(rb0, oB_h.at[pl.ds(b0, B)], s5)
            dA1.wait()
            wA1 = pltpu.async_copy(ra1, oA_h.at[pl.ds(b1, B)], s6)
            dB1.wait()
            wB1 = pltpu.async_copy(rb1, oB_h.at[pl.ds(b1, B)], s7)
            wA0.wait()
            wB0.wait()
            wA1.wait()
            wB1.wait()
            return 0
        lax.fori_loop(0, nb2, body, 0)

    return k(tA, iA, tB, iB)


def _sc_min(table, src, dst):
    B = 160
    EPC = NNZ // 4

    @functools.partial(
        pl.kernel,
        out_type=jax.ShapeDtypeStruct((4, 8, EH * 16), jnp.float32),
        mesh=_mesh(),
        scratch_types=[pltpu.VMEM((B,), jnp.int32),
                       pltpu.VMEM((B,), jnp.int32),
                       pltpu.VMEM((B // 2, HID), jnp.float32),
                       pltpu.VMEM((B // 2, HID), jnp.float32),
                       pltpu.VMEM((EH * 16,), jnp.float32),
                       pltpu.SemaphoreType.DMA,
                       pltpu.SemaphoreType.DMA],
    )
    def k(t_h, src_h, dst_h, out_h, srcb, dstb, rows0, rows1, acc, sem0, sem1):
        cid = lax.axis_index("c")
        sid = lax.axis_index("s")
        wid = sid * NC + cid
        c = wid % 4
        g = wid // 4

        def ini(j, _):
            acc[pl.ds(j * 16, 16)] = jnp.full((16,), jnp.inf, jnp.float32)
            return 0
        lax.fori_loop(0, EH, ini, 0)

        def update(rows, dstb_off):
            def upd(t, _):
                dvec = dstb[pl.ds(dstb_off + 16 * t, 16)]
                for j in range(16):
                    d = dvec[j]
                    rrow = rows.at[t * 16 + j]
                    acc[pl.ds(d * 16, 16)] = jnp.minimum(
                        acc[pl.ds(d * 16, 16)], rrow[pl.ds(16 * g, 16)])
                return 0
            lax.fori_loop(0, (B // 2) // 16, upd, 0)

        def body(i, _):
            base = c * EPC + i * B
            pltpu.sync_copy(src_h.at[pl.ds(base, B)], srcb)
            pltpu.sync_copy(dst_h.at[pl.ds(base, B)], dstb)
            d0 = pltpu.async_copy(t_h.at[srcb.at[pl.ds(0, B // 2)]], rows0, sem0)
            d1 = pltpu.async_copy(t_h.at[srcb.at[pl.ds(B // 2, B // 2)]],
                                  rows1, sem1)
            d0.wait()
            update(rows0, 0)
            d1.wait()
            update(rows1, B // 2)
            return 0
        lax.fori_loop(0, EPC // B, body, 0)
        pltpu.sync_copy(acc, out_h.at[c, g])

    return k(table, src, dst)


def _single(body, out_shape, *args):
    return pl.pallas_call(body, out_shape=out_shape)(*args)


def _tc_mmt(x, W, b):
    R, K = x.shape
    blk = 2000
    def body(x_ref, w_ref, b_ref, o_ref):
        o_ref[...] = jnp.tanh(x_ref[...] @ w_ref[...] + b_ref[...])
    return pl.pallas_call(
        body,
        grid=(R // blk,),
        in_specs=[pl.BlockSpec((blk, K), lambda i: (i, 0)),
                  pl.BlockSpec((K, HID), lambda i: (0, 0)),
                  pl.BlockSpec((1, HID), lambda i: (0, 0))],
        out_specs=pl.BlockSpec((blk, HID), lambda i: (i, 0)),
        out_shape=jax.ShapeDtypeStruct((R, HID), jnp.float32),
    )(x, W, b.reshape(1, HID))


def _tc_merge(parts, deg2):
    U = parts.shape[1]
    def body(p_ref, d_ref, o_ref):
        p = p_ref[...]
        d = d_ref[...]
        deg = jnp.clip(d[0] + d[1], 1.0, None)
        o_ref[...] = (p[0] + p[1]) / deg
    return _single(body, jax.ShapeDtypeStruct((U, HID), jnp.float32),
                   parts, deg2.reshape(NC, U, 1))


def _tc_merge_mmt(parts, deg2, W, b):
    U = parts.shape[1]
    def body(p_ref, d_ref, w_ref, b_ref, o_ref):
        p = p_ref[...]
        d = d_ref[...]
        deg = jnp.clip(d[0] + d[1], 1.0, None)
        h = (p[0] + p[1]) / deg
        o_ref[...] = jnp.tanh(h @ w_ref[...] + b_ref[...])
    return _single(body, jax.ShapeDtypeStruct((U, HID), jnp.float32),
                   parts, deg2.reshape(NC, U, 1), W, b.reshape(1, HID))


def _tc_he(x_e, w1, b1, w2, b2):
    def body(x_ref, w1_ref, b1_ref, w2_ref, b2_ref, o_ref):
        h = jnp.tanh(x_ref[...] @ w1_ref[...] + b1_ref[...])
        o_ref[...] = jnp.tanh(h @ w2_ref[...] + b2_ref[...])
    return _single(body, jax.ShapeDtypeStruct((EH, HID), jnp.float32),
                   x_e, w1, b1.reshape(1, HID), w2, b2.reshape(1, HID))


def _tc_qkv(h, he, p):
    def body(h_ref, he_ref, wqe, wkn, wvn, wqn, wke, wve,
             qe_r, kn_r, vn_r, qn_r, ke_r, ve_r):
        hv = h_ref[...]
        hev = he_ref[...]
        qe_r[...] = hev @ wqe[...]
        ke_r[...] = hev @ wke[...]
        ve_r[...] = hev @ wve[...]
        kn_r[...] = hv @ wkn[...]
        vn_r[...] = hv @ wvn[...]
        qn_r[...] = hv @ wqn[...]
    sh_e = jax.ShapeDtypeStruct((EH, HID), jnp.float32)
    sh_n = jax.ShapeDtypeStruct((N, HID), jnp.float32)
    return pl.pallas_call(
        body, out_shape=(sh_e, sh_n, sh_n, sh_n, sh_e, sh_e),
    )(h, he, p['Wqe'], p['Wkn'], p['Wvn'], p['Wqn'], p['Wke'], p['Wve'])


def _tc_dotexp_scale(G1, G2, Gv):
    blk = 3200
    inv_d = 1.0 / (float(HID) ** 0.5)

    def body(g1_ref, g2_ref, gv_ref, e_ref, sg_ref):
        s = jnp.sum(g1_ref[...] * g2_ref[...], axis=1, keepdims=True)
        e = jnp.exp(s * inv_d)
        e_ref[...] = e
        sg_ref[...] = e * gv_ref[...]
    return pl.pallas_call(
        body,
        grid=(NNZ // blk,),
        in_specs=[pl.BlockSpec((blk, HID), lambda i: (i, 0)),
                  pl.BlockSpec((blk, HID), lambda i: (i, 0)),
                  pl.BlockSpec((blk, HID), lambda i: (i, 0))],
        out_specs=[pl.BlockSpec((blk, 1), lambda i: (i, 0)),
                   pl.BlockSpec((blk, HID), lambda i: (i, 0))],
        out_shape=(jax.ShapeDtypeStruct((NNZ, 1), jnp.float32),
                   jax.ShapeDtypeStruct((NNZ, HID), jnp.float32)),
    )(G1, G2, Gv)


def _tc_attn_post(parts, zparts, x):
    U = x.shape[0]

    def body(p_ref, z_ref, x_ref, o_ref):
        p = p_ref[...]
        zp = z_ref[...]
        s = p[0] + p[1]
        z = zp[0] + zp[1]
        o_ref[...] = jnp.tanh(x_ref[...] + s / (z + 1e-16))
    return _single(body, jax.ShapeDtypeStruct((U, HID), jnp.float32),
                   parts, zparts.reshape(NC, U, 1), x)


def _bn_in(h, g, b):
    mu = jnp.mean(h, axis=0, keepdims=True)
    var = jnp.mean((h - mu) * (h - mu), axis=0, keepdims=True)
    return (h - mu) / jnp.sqrt(var + 1e-5) * g + b


def _tc_nf(h, hs, p):
    def body(h_ref, hs_ref, g1, be1, w, b, g2, be2, o_ref):
        hc = jnp.concatenate([h_ref[...], hs_ref[...]], axis=1)
        hc = _bn_in(hc, g1[...], be1[...])
        hc = jnp.tanh(hc @ w[...] + b[...])
        o_ref[...] = _bn_in(hc, g2[...], be2[...])
    return _single(body, jax.ShapeDtypeStruct((N, HID), jnp.float32),
                   h, hs, p['g1'].reshape(1, 2 * HID), p['be1'].reshape(1, 2 * HID),
                   p['W'], p['b'].reshape(1, HID),
                   p['g2'].reshape(1, HID), p['be2'].reshape(1, HID))


def _tc_minmerge(minp):
    def body(mp_ref, o_ref):
        o_ref[...] = jnp.min(mp_ref[...], axis=0)
    return _single(body, jax.ShapeDtypeStruct((8, EH * 16 // 128, 128), jnp.float32),
                   minp.reshape(4, 8, EH * 16 // 128, 128))


def _tc_ef_cls(xa8, he, pe, pc):
    def body(mp_ref, he_ref, g1, be1, w, b, g2, be2, w1, b1, w2, b2, o_ref):
        m = mp_ref[...]
        xa = jnp.concatenate([m[i] for i in range(8)], axis=1)
        xa = jnp.where(jnp.isfinite(xa), xa, 0.0)
        hf = jnp.concatenate([xa, he_ref[...]], axis=1)
        hf = _bn_in(hf, g1[...], be1[...])
        hf = jnp.tanh(hf @ w[...] + b[...])
        hf = _bn_in(hf, g2[...], be2[...])
        o_ref[...] = jnp.tanh(hf @ w1[...] + b1[...]) @ w2[...] + b2[...]
    return _single(body, jax.ShapeDtypeStruct((EH, OUT_C), jnp.float32),
                   xa8, he,
                   pe['g1'].reshape(1, 2 * HID), pe['be1'].reshape(1, 2 * HID),
                   pe['W'], pe['b'].reshape(1, HID),
                   pe['g2'].reshape(1, HID), pe['be2'].reshape(1, HID),
                   pc['W1'], pc['b1'].reshape(1, HID),
                   pc['W2'], pc['b2'].reshape(1, OUT_C))


def kernel(x, x_struct, x_e, edge_index, params):
    src = edge_index[0]
    dst = edge_index[1]

    de, dn = _sc_hist(src, dst)

    def hconv_chain(t):
        ep = _sc_gs(t, src, dst, EH, HID)
        e = _tc_merge(ep, de)
        nparts = _sc_gs(e, dst, src, N, HID)
        return nparts

    psem, pstr = params['sem'], params['str']
    t_sem = _tc_mmt(x, psem['W1'], psem['b1'])
    t_str = _tc_mmt(x_struct, pstr['W1'], pstr['b1'])
    t2_sem = _tc_merge_mmt(hconv_chain(t_sem), dn, psem['W2'], psem['b2'])
    t2_str = _tc_merge_mmt(hconv_chain(t_str), dn, pstr['W2'], pstr['b2'])
    h = _tc_merge(hconv_chain(t2_sem), dn)
    hs = _tc_merge(hconv_chain(t2_str), dn)

    phe = params['he']
    he = _tc_he(x_e, phe['W1'], phe['b1'], phe['W2'], phe['b2'])

    for i in range(3):
        p = params['att'][i]
        qe, kn, vn, qn, ke, ve = _tc_qkv(h, he, p)
        G1, G2, Gv = _sc_gather3(qe, dst, kn, src, vn, src)
        e1s, SG1 = _tc_dotexp_scale(G1, G2, Gv)
        P1, Z1 = _sc_scatter(SG1, e1s.reshape(NNZ), dst, EH)
        he = _tc_attn_post(P1, Z1, he)
        G3, G4, Ge = _sc_gather3(qn, src, ke, dst, ve, dst)
        e2s, SG2 = _tc_dotexp_scale(G3, G4, Ge)
        P2, Z2 = _sc_scatter(SG2, e2s.reshape(NNZ), src, N)
        h = _tc_attn_post(P2, Z2, h)

    hc = _tc_nf(h, hs, params['nf'])
    minp = _sc_min(hc, src, dst)
    xa8 = _tc_minmerge(minp).reshape(8, EH, 16)
    return _tc_ef_cls(xa8, he, params['ef'], params['cls'])


# speedup vs baseline: 1.1303x; 1.1303x over previous
import functools

import jax
import jax.numpy as jnp
from jax import lax
from jax.experimental import pallas as pl
from jax.experimental.pallas import tpu as pltpu
from jax.experimental.pallas import tpu_sc as plsc

N = 10000
EH = 5000
NNZ = 320000
HID = 128
OUT_C = 16

NC = 2
NS = 16
NW = NC * NS
EPW = NNZ // NW


def _mesh():
    return plsc.VectorSubcoreMesh(core_axis_name="c", subcore_axis_name="s")


_CH = 200


def _zero_acc(z_h, acc, sid, U):
    def zc(k, _):
        @pl.when(k % NS == sid)
        def _():
            pltpu.sync_copy(z_h.at[pl.ds(k * _CH, _CH)],
                            acc.at[pl.ds(k * _CH, _CH)])
        return 0
    lax.fori_loop(0, U // _CH, zc, 0)


def _acc_out(acc, out_h, cid, sid, U):
    def oc(k, _):
        @pl.when(k % NS == sid)
        def _():
            pltpu.sync_copy(acc.at[pl.ds(k * _CH, _CH)],
                            out_h.at[cid, pl.ds(k * _CH, _CH)])
        return 0
    lax.fori_loop(0, U // _CH, oc, 0)


def _sc_hist(src, dst):
    B = 400

    @functools.partial(
        pl.kernel,
        out_type=(jax.ShapeDtypeStruct((NC, EH), jnp.float32),
                  jax.ShapeDtypeStruct((NC, N), jnp.float32)),
        mesh=_mesh(),
        scratch_types=[pltpu.VMEM((B,), jnp.int32),
                       pltpu.VMEM((B,), jnp.int32),
                       pltpu.VMEM((B,), jnp.float32),
                       pltpu.VMEM_SHARED((EH,), jnp.float32),
                       pltpu.VMEM_SHARED((N,), jnp.float32)],
    )
    def k(src_h, dst_h, zeh_h, zn_h, de_h, dn_h, srcb, dstb, onesb, eacc, nacc):
        cid = lax.axis_index("c")
        sid = lax.axis_index("s")
        wid = sid * NC + cid

        @pl.when(sid == 0)
        def _():
            pltpu.sync_copy(zeh_h, eacc)
            pltpu.sync_copy(zn_h, nacc)

        def fill(i, _):
            onesb[pl.ds(i * 16, 16)] = jnp.full((16,), 1.0, jnp.float32)
            return 0
        lax.fori_loop(0, B // 16, fill, 0)
        plsc.subcore_barrier()

        def body(i, _):
            base = wid * EPW + i * B
            pltpu.sync_copy(dst_h.at[pl.ds(base, B)], dstb)
            pltpu.sync_copy(src_h.at[pl.ds(base, B)], srcb)
            pltpu.sync_copy(onesb, eacc.at[dstb], add=True)
            pltpu.sync_copy(onesb, nacc.at[srcb], add=True)
            return 0
        lax.fori_loop(0, EPW // B, body, 0)
        plsc.subcore_barrier()

        @pl.when(sid == 0)
        def _():
            pltpu.sync_copy(eacc, de_h.at[cid])
            pltpu.sync_copy(nacc, dn_h.at[cid])

    return k(src, dst, jnp.zeros((EH,), jnp.float32), jnp.zeros((N,), jnp.float32))


def _sc_gather(table, idx, W):
    B = 51200 // W
    nb = EPW // B

    @functools.partial(
        pl.kernel,
        out_type=jax.ShapeDtypeStruct((NNZ, W), jnp.float32),
        mesh=_mesh(),
        scratch_types=[pltpu.VMEM((B,), jnp.int32),
                       pltpu.VMEM((B, W), jnp.float32),
                       pltpu.SemaphoreType.DMA],
    )
    def k(table_h, idx_h, out_h, idxb, rows, sem):
        cid = lax.axis_index("c")
        sid = lax.axis_index("s")
        wid = sid * NC + cid

        def body(i, _):
            base = wid * EPW + i * B
            pltpu.sync_copy(idx_h.at[pl.ds(base, B)], idxb)
            pltpu.async_copy(table_h.at[idxb], rows, sem).wait()
            pltpu.sync_copy(rows, out_h.at[pl.ds(base, B)])
            return 0
        lax.fori_loop(0, nb, body, 0)

    return k(table, idx)


def _sc_gs(table, gidx, didx, U, W, scale=None):
    scaled = scale is not None
    B = 200 if (U == EH and not scaled) else 80
    nb = EPW // B
    outs = [jax.ShapeDtypeStruct((NC, U, W), jnp.float32)]
    scr = ([pltpu.VMEM((B,), jnp.int32)] * 4 +
           [pltpu.VMEM((B, W), jnp.float32)] * 2 +
           [pltpu.VMEM_SHARED((U, W), jnp.float32)] +
           [pltpu.SemaphoreType.DMA] * 2)
    ins = [table, gidx, didx, jnp.zeros((U, W), jnp.float32)]
    if scaled:
        outs.append(jax.ShapeDtypeStruct((NC, U), jnp.float32))
        scr += [pltpu.VMEM((B,), jnp.float32)] * 2 + \
               [pltpu.VMEM_SHARED((U,), jnp.float32)]
        ins += [jnp.zeros((U,), jnp.float32), scale]

    @functools.partial(pl.kernel,
                       out_type=tuple(outs) if scaled else outs[0],
                       mesh=_mesh(), scratch_types=scr)
    def k(*refs):
        if scaled:
            (table_h, gidx_h, didx_h, z_h, zu_h, sc_h, out_h, zout_h,
             gb0, gb1, db0, db1, rows0, rows1, acc, sem0, sem1,
             scb0, scb1, zacc) = refs
        else:
            (table_h, gidx_h, didx_h, z_h, out_h,
             gb0, gb1, db0, db1, rows0, rows1, acc, sem0, sem1) = refs
            sc_h = zu_h = zout_h = scb0 = scb1 = zacc = None
        cid = lax.axis_index("c")
        sid = lax.axis_index("s")
        wid = sid * NC + cid

        _zero_acc(z_h, acc, sid, U)
        if scaled:
            @pl.when(sid == 0)
            def _():
                pltpu.sync_copy(zu_h, zacc)
        plsc.subcore_barrier()

        def start(base, gb, db, scb, rows, sem):
            pltpu.sync_copy(gidx_h.at[pl.ds(base, B)], gb)
            pltpu.sync_copy(didx_h.at[pl.ds(base, B)], db)
            if scaled:
                pltpu.sync_copy(sc_h.at[pl.ds(base, B)], scb)
            return pltpu.async_copy(table_h.at[gb], rows, sem)

        def finish(d, db, scb, rows):
            d.wait()
            if scaled:
                def mule(t, _):
                    svec = scb[pl.ds(16 * t, 16)]
                    for j in range(16):
                        sv = svec[j]
                        row = rows.at[t * 16 + j]
                        for v in range(W // 16):
                            row[pl.ds(16 * v, 16)] = row[pl.ds(16 * v, 16)] * sv
                    return 0
                lax.fori_loop(0, B // 16, mule, 0)
            pltpu.sync_copy(rows, acc.at[db], add=True)
            if scaled:
                pltpu.sync_copy(scb, zacc.at[db], add=True)

        def body(j, _):
            b0 = wid * EPW + (2 * j) * B
            d0 = start(b0, gb0, db0, scb0, rows0, sem0)
            d1 = start(b0 + B, gb1, db1, scb1, rows1, sem1)
            finish(d0, db0, scb0, rows0)
            finish(d1, db1, scb1, rows1)
            return 0
        lax.fori_loop(0, nb // 2, body, 0)
        if nb % 2:
            bl = wid * EPW + (nb - 1) * B
            finish(start(bl, gb0, db0, scb0, rows0, sem0), db0, scb0, rows0)

        plsc.subcore_barrier()
        _acc_out(acc, out_h, cid, sid, U)
        if scaled:
            @pl.when(sid == 0)
            def _():
                pltpu.sync_copy(zacc, zout_h.at[cid])

    return k(*ins)


def _sc_gather2(tA, iA, tB, iB):
    B = 200
    nb2 = EPW // (2 * B)
    sh = jax.ShapeDtypeStruct((NNZ, HID), jnp.float32)

    @functools.partial(
        pl.kernel, out_type=(sh, sh), mesh=_mesh(),
        scratch_types=[pltpu.VMEM((B,), jnp.int32)] * 4 +
                      [pltpu.VMEM((B, HID), jnp.float32)] * 4 +
                      [pltpu.SemaphoreType.DMA] * 8,
    )
    def k(tA_h, iA_h, tB_h, iB_h, oA_h, oB_h,
          ia0, ia1, ib0, ib1, ra0, ra1, rb0, rb1,
          s0, s1, s2, s3, s4, s5, s6, s7):
        cid = lax.axis_index("c")
        sid = lax.axis_index("s")
        wid = sid * NC + cid

        def body(j, _):
            b0 = wid * EPW + (2 * j) * B
            b1 = b0 + B
            pltpu.sync_copy(iA_h.at[pl.ds(b0, B)], ia0)
            pltpu.sync_copy(iB_h.at[pl.ds(b0, B)], ib0)
            pltpu.sync_copy(iA_h.at[pl.ds(b1, B)], ia1)
            pltpu.sync_copy(iB_h.at[pl.ds(b1, B)], ib1)
            dA0 = pltpu.async_copy(tA_h.at[ia0], ra0, s0)
            dB0 = pltpu.async_copy(tB_h.at[ib0], rb0, s1)
            dA1 = pltpu.async_copy(tA_h.at[ia1], ra1, s2)
            dB1 = pltpu.async_copy(tB_h.at[ib1], rb1, s3)
            dA0.wait()
            wA0 = pltpu.async_copy(ra0, oA_h.at[pl.ds(b0, B)], s4)
            dB0.wait()
            wB0 = pltpu.async_copy(rb0, oB_h.at[pl.ds(b0, B)], s5)
            dA1.wait()
            wA1 = pltpu.async_copy(ra1, oA_h.at[pl.ds(b1, B)], s6)
            dB1.wait()
            wB1 = pltpu.async_copy(rb1, oB_h.at[pl.ds(b1, B)], s7)
            wA0.wait()
            wB0.wait()
            wA1.wait()
            wB1.wait()
            return 0
        lax.fori_loop(0, nb2, body, 0)

    return k(tA, iA, tB, iB)


def _sc_min(table, src, dst):
    B = 320
    EPC = NNZ // 4

    @functools.partial(
        pl.kernel,
        out_type=jax.ShapeDtypeStruct((4, 8, EH * 16), jnp.float32),
        mesh=_mesh(),
        scratch_types=[pltpu.VMEM((B,), jnp.int32),
                       pltpu.VMEM((B,), jnp.int32),
                       pltpu.VMEM((B // 2, HID), jnp.float32),
                       pltpu.VMEM((B // 2, HID), jnp.float32),
                       pltpu.VMEM((EH * 16,), jnp.float32),
                       pltpu.SemaphoreType.DMA,
                       pltpu.SemaphoreType.DMA],
    )
    def k(t_h, src_h, dst_h, out_h, srcb, dstb, rows0, rows1, acc, sem0, sem1):
        cid = lax.axis_index("c")
        sid = lax.axis_index("s")
        wid = sid * NC + cid
        c = wid % 4
        g = wid // 4

        def ini(j, _):
            acc[pl.ds(j * 16, 16)] = jnp.full((16,), jnp.inf, jnp.float32)
            return 0
        lax.fori_loop(0, EH, ini, 0)

        def update(rows, dstb_off):
            def upd(t, _):
                dvec = dstb[pl.ds(dstb_off + 16 * t, 16)]
                for j in range(16):
                    d = dvec[j]
                    rrow = rows.at[t * 16 + j]
                    acc[pl.ds(d * 16, 16)] = jnp.minimum(
                        acc[pl.ds(d * 16, 16)], rrow[pl.ds(16 * g, 16)])
                return 0
            lax.fori_loop(0, (B // 2) // 16, upd, 0)

        def body(i, _):
            base = c * EPC + i * B
            pltpu.sync_copy(src_h.at[pl.ds(base, B)], srcb)
            pltpu.sync_copy(dst_h.at[pl.ds(base, B)], dstb)
            d0 = pltpu.async_copy(t_h.at[srcb.at[pl.ds(0, B // 2)]], rows0, sem0)
            d1 = pltpu.async_copy(t_h.at[srcb.at[pl.ds(B // 2, B // 2)]],
                                  rows1, sem1)
            d0.wait()
            update(rows0, 0)
            d1.wait()
            update(rows1, B // 2)
            return 0
        lax.fori_loop(0, EPC // B, body, 0)
        pltpu.sync_copy(acc, out_h.at[c, g])

    return k(table, src, dst)


def _single(body, out_shape, *args):
    return pl.pallas_call(body, out_shape=out_shape)(*args)


def _tc_mmt(x, W, b):
    R, K = x.shape
    blk = 2000
    def body(x_ref, w_ref, b_ref, o_ref):
        o_ref[...] = jnp.tanh(x_ref[...] @ w_ref[...] + b_ref[...])
    return pl.pallas_call(
        body,
        grid=(R // blk,),
        in_specs=[pl.BlockSpec((blk, K), lambda i: (i, 0)),
                  pl.BlockSpec((K, HID), lambda i: (0, 0)),
                  pl.BlockSpec((1, HID), lambda i: (0, 0))],
        out_specs=pl.BlockSpec((blk, HID), lambda i: (i, 0)),
        out_shape=jax.ShapeDtypeStruct((R, HID), jnp.float32),
    )(x, W, b.reshape(1, HID))


def _tc_merge(parts, deg2):
    U = parts.shape[1]
    def body(p_ref, d_ref, o_ref):
        p = p_ref[...]
        d = d_ref[...]
        deg = jnp.clip(d[0] + d[1], 1.0, None)
        o_ref[...] = (p[0] + p[1]) / deg
    return _single(body, jax.ShapeDtypeStruct((U, HID), jnp.float32),
                   parts, deg2.reshape(NC, U, 1))


def _tc_merge_mmt(parts, deg2, W, b):
    U = parts.shape[1]
    def body(p_ref, d_ref, w_ref, b_ref, o_ref):
        p = p_ref[...]
        d = d_ref[...]
        deg = jnp.clip(d[0] + d[1], 1.0, None)
        h = (p[0] + p[1]) / deg
        o_ref[...] = jnp.tanh(h @ w_ref[...] + b_ref[...])
    return _single(body, jax.ShapeDtypeStruct((U, HID), jnp.float32),
                   parts, deg2.reshape(NC, U, 1), W, b.reshape(1, HID))


def _tc_he(x_e, w1, b1, w2, b2):
    def body(x_ref, w1_ref, b1_ref, w2_ref, b2_ref, o_ref):
        h = jnp.tanh(x_ref[...] @ w1_ref[...] + b1_ref[...])
        o_ref[...] = jnp.tanh(h @ w2_ref[...] + b2_ref[...])
    return _single(body, jax.ShapeDtypeStruct((EH, HID), jnp.float32),
                   x_e, w1, b1.reshape(1, HID), w2, b2.reshape(1, HID))


def _tc_qkv(h, he, p):
    def body(h_ref, he_ref, wqe, wkn, wvn, wqn, wke, wve,
             qe_r, kn_r, vn_r, qn_r, ke_r, ve_r):
        hv = h_ref[...]
        hev = he_ref[...]
        qe_r[...] = hev @ wqe[...]
        ke_r[...] = hev @ wke[...]
        ve_r[...] = hev @ wve[...]
        kn_r[...] = hv @ wkn[...]
        vn_r[...] = hv @ wvn[...]
        qn_r[...] = hv @ wqn[...]
    sh_e = jax.ShapeDtypeStruct((EH, HID), jnp.float32)
    sh_n = jax.ShapeDtypeStruct((N, HID), jnp.float32)
    return pl.pallas_call(
        body, out_shape=(sh_e, sh_n, sh_n, sh_n, sh_e, sh_e),
    )(h, he, p['Wqe'], p['Wkn'], p['Wvn'], p['Wqn'], p['Wke'], p['Wve'])


def _tc_dotexp(G1, G2):
    blk = 3200
    inv_d = 1.0 / (float(HID) ** 0.5)

    def body(g1_ref, g2_ref, o_ref):
        s = jnp.sum(g1_ref[...] * g2_ref[...], axis=1, keepdims=True)
        o_ref[...] = jnp.exp(s * inv_d)
    return pl.pallas_call(
        body,
        grid=(NNZ // blk,),
        in_specs=[pl.BlockSpec((blk, HID), lambda i: (i, 0)),
                  pl.BlockSpec((blk, HID), lambda i: (i, 0))],
        out_specs=pl.BlockSpec((blk, 1), lambda i: (i, 0)),
        out_shape=jax.ShapeDtypeStruct((NNZ, 1), jnp.float32),
    )(G1, G2)


def _tc_attn_post(parts, zparts, x):
    U = x.shape[0]

    def body(p_ref, z_ref, x_ref, o_ref):
        p = p_ref[...]
        zp = z_ref[...]
        s = p[0] + p[1]
        z = zp[0] + zp[1]
        o_ref[...] = jnp.tanh(x_ref[...] + s / (z + 1e-16))
    return _single(body, jax.ShapeDtypeStruct((U, HID), jnp.float32),
                   parts, zparts.reshape(NC, U, 1), x)


def _bn_in(h, g, b):
    mu = jnp.mean(h, axis=0, keepdims=True)
    var = jnp.mean((h - mu) * (h - mu), axis=0, keepdims=True)
    return (h - mu) / jnp.sqrt(var + 1e-5) * g + b


def _tc_nf(h, hs, p):
    def body(h_ref, hs_ref, g1, be1, w, b, g2, be2, o_ref):
        hc = jnp.concatenate([h_ref[...], hs_ref[...]], axis=1)
        hc = _bn_in(hc, g1[...], be1[...])
        hc = jnp.tanh(hc @ w[...] + b[...])
        o_ref[...] = _bn_in(hc, g2[...], be2[...])
    return _single(body, jax.ShapeDtypeStruct((N, HID), jnp.float32),
                   h, hs, p['g1'].reshape(1, 2 * HID), p['be1'].reshape(1, 2 * HID),
                   p['W'], p['b'].reshape(1, HID),
                   p['g2'].reshape(1, HID), p['be2'].reshape(1, HID))


def _tc_minmerge(minp):
    def body(mp_ref, o_ref):
        o_ref[...] = jnp.min(mp_ref[...], axis=0)
    return _single(body, jax.ShapeDtypeStruct((8, EH * 16 // 128, 128), jnp.float32),
                   minp.reshape(4, 8, EH * 16 // 128, 128))


def _tc_ef_cls(xa8, he, pe, pc):
    def body(mp_ref, he_ref, g1, be1, w, b, g2, be2, w1, b1, w2, b2, o_ref):
        m = mp_ref[...]
        xa = jnp.concatenate([m[i] for i in range(8)], axis=1)
        xa = jnp.where(jnp.isfinite(xa), xa, 0.0)
        hf = jnp.concatenate([xa, he_ref[...]], axis=1)
        hf = _bn_in(hf, g1[...], be1[...])
        hf = jnp.tanh(hf @ w[...] + b[...])
        hf = _bn_in(hf, g2[...], be2[...])
        o_ref[...] = jnp.tanh(hf @ w1[...] + b1[...]) @ w2[...] + b2[...]
    return _single(body, jax.ShapeDtypeStruct((EH, OUT_C), jnp.float32),
                   xa8, he,
                   pe['g1'].reshape(1, 2 * HID), pe['be1'].reshape(1, 2 * HID),
                   pe['W'], pe['b'].reshape(1, HID),
                   pe['g2'].reshape(1, HID), pe['be2'].reshape(1, HID),
                   pc['W1'], pc['b1'].reshape(1, HID),
                   pc['W2'], pc['b2'].reshape(1, OUT_C))


def kernel(x, x_struct, x_e, edge_index, params):
    src = edge_index[0]
    dst = edge_index[1]

    de, dn = _sc_hist(src, dst)

    def hconv_chain(t):
        ep = _sc_gs(t, src, dst, EH, HID)
        e = _tc_merge(ep, de)
        nparts = _sc_gs(e, dst, src, N, HID)
        return nparts

    psem, pstr = params['sem'], params['str']
    t_sem = _tc_mmt(x, psem['W1'], psem['b1'])
    t_str = _tc_mmt(x_struct, pstr['W1'], pstr['b1'])
    t2_sem = _tc_merge_mmt(hconv_chain(t_sem), dn, psem['W2'], psem['b2'])
    t2_str = _tc_merge_mmt(hconv_chain(t_str), dn, pstr['W2'], pstr['b2'])
    h = _tc_merge(hconv_chain(t2_sem), dn)
    hs = _tc_merge(hconv_chain(t2_str), dn)

    phe = params['he']
    he = _tc_he(x_e, phe['W1'], phe['b1'], phe['W2'], phe['b2'])

    for i in range(3):
        p = params['att'][i]
        qe, kn, vn, qn, ke, ve = _tc_qkv(h, he, p)
        G1, G2 = _sc_gather2(qe, dst, kn, src)
        e1s = _tc_dotexp(G1, G2).reshape(NNZ)
        P1, Z1 = _sc_gs(vn, src, dst, EH, HID, scale=e1s)
        he = _tc_attn_post(P1, Z1, he)
        G3, G4 = _sc_gather2(qn, src, ke, dst)
        e2s = _tc_dotexp(G3, G4).reshape(NNZ)
        P2, Z2 = _sc_gs(ve, dst, src, N, HID, scale=e2s)
        h = _tc_attn_post(P2, Z2, h)

    hc = _tc_nf(h, hs, params['nf'])
    minp = _sc_min(hc, src, dst)
    xa8 = _tc_minmerge(minp).reshape(8, EH, 16)
    return _tc_ef_cls(xa8, he, params['ef'], params['cls'])
